# Initial kernel scaffold; baseline (speedup 1.0000x reference)
import jax
import jax.numpy as jnp
from jax.experimental import pallas as pl


def kernel(x, edge_index, W0a, b0a, W0b, b0b, Wm1, bm1, Wm2, bm2, Wlast):
    raise NotImplementedError("write your pallas kernel here")


# speedup vs baseline: 7.5352x; 7.5352x over previous
import functools

import jax
import jax.numpy as jnp
from jax import lax
from jax.experimental import pallas as pl
from jax.experimental.pallas import tpu as pltpu
from jax.experimental.pallas import tpu_sc as plsc

_N = 10000
_E = 320000
_NCORE = 2
_NSUB = 16
_NW = _NCORE * _NSUB
_CH = 128
_CHUNKS = _E // _CH
_BASE_CH = _CHUNKS // _NW
_EXTRA = _CHUNKS - _BASE_CH * _NW
_NPAD = 10240
_RPT = _NPAD // _NSUB

_SELU_ALPHA = 1.6732632423543772
_SELU_SCALE = 1.0507009873554805


def _selu(v):
    return _SELU_SCALE * jnp.where(v > 0, v, _SELU_ALPHA * (jnp.exp(v) - 1.0))


def _make_agg(width, interpret=False):
    mesh = plsc.VectorSubcoreMesh(
        core_axis_name="c", subcore_axis_name="s",
        num_cores=_NCORE, num_subcores=_NSUB,
    )

    @functools.partial(
        pl.kernel,
        out_type=jax.ShapeDtypeStruct((_NCORE, _NPAD, width), jnp.float32),
        mesh=mesh,
        scratch_types=[
            pltpu.VMEM((_CH,), jnp.int32),
            pltpu.VMEM((_CH,), jnp.int32),
            pltpu.VMEM((_CH, width), jnp.float32),
            pltpu.VMEM((_RPT, width), jnp.float32),
            pltpu.VMEM_SHARED((_NPAD, width), jnp.float32),
            pltpu.SemaphoreType.DMA,
        ],
        compiler_params=pltpu.CompilerParams(use_tc_tiling_on_sc=False),
        interpret=interpret,
    )
    def agg(y_hbm, src_hbm, dst_hbm, out_hbm, idx_s, idx_d, rows, zbuf, acc, sem):
        cid = lax.axis_index("c")
        sid = lax.axis_index("s")
        wid = sid * _NCORE + cid

        def zrow(j, carry):
            for w in range(width // 16):
                zbuf[j, pl.ds(w * 16, 16)] = jnp.zeros((16,), jnp.float32)
            return carry

        lax.fori_loop(0, _RPT, zrow, 0)
        row0 = sid * _RPT
        pltpu.sync_copy(zbuf, acc.at[pl.ds(row0, _RPT)])
        plsc.subcore_barrier()

        nchunks = _BASE_CH + jnp.where(wid < _EXTRA, 1, 0)
        chunk0 = wid * _BASE_CH + jnp.minimum(wid, _EXTRA)

        def body(j, carry):
            e0 = pl.multiple_of((chunk0 + j) * _CH, _CH)
            pltpu.sync_copy(src_hbm.at[pl.ds(e0, _CH)], idx_s)
            pltpu.sync_copy(dst_hbm.at[pl.ds(e0, _CH)], idx_d)
            pltpu.async_copy(y_hbm.at[idx_s], rows, sem).wait()
            pltpu.sync_copy(rows, acc.at[idx_d], add=True)
            return carry

        lax.fori_loop(0, nchunks, body, 0)
        plsc.subcore_barrier()
        pltpu.sync_copy(
            acc.at[pl.ds(row0, _RPT)], out_hbm.at[cid, pl.ds(row0, _RPT)]
        )

    return agg


def _proj0_body(x_ref, w_ref, y_ref):
    y_ref[...] = jnp.dot(
        x_ref[...], w_ref[...], preferred_element_type=jnp.float32
    )


def _node0_body(a_ref, y_ref, b0a_ref, w0b_ref, b0b_ref, wm1_ref, x_ref, yn_ref):
    pre = a_ref[0, :_N] + a_ref[1, :_N] + y_ref[...] + b0a_ref[...]
    x1 = (
        jnp.dot(_selu(pre), w0b_ref[...], preferred_element_type=jnp.float32)
        + b0b_ref[...]
    )
    x_ref[...] = x1
    yn_ref[...] = jnp.dot(x1, wm1_ref[...], preferred_element_type=jnp.float32)


def _node_mid_body(
    x_ref, a_ref, y_ref, b1_ref, w2_ref, b2_ref, wn_ref, xo_ref, yn_ref
):
    pre = a_ref[0, :_N] + a_ref[1, :_N] + y_ref[...] + b1_ref[...]
    h = (
        jnp.dot(_selu(pre), w2_ref[...], preferred_element_type=jnp.float32)
        + b2_ref[...]
    )
    xn = x_ref[...] + h
    xo_ref[...] = xn
    yn_ref[...] = jnp.dot(xn, wn_ref[...], preferred_element_type=jnp.float32)


def _node_last_body(x_ref, a_ref, y_ref, b1_ref, w2_ref, b2_ref, xo_ref):
    pre = a_ref[0, :_N] + a_ref[1, :_N] + y_ref[...] + b1_ref[...]
    h = (
        jnp.dot(_selu(pre), w2_ref[...], preferred_element_type=jnp.float32)
        + b2_ref[...]
    )
    xo_ref[...] = x_ref[...] + h


def _final_body(x_ref, a_ref, wl_ref, o_ref):
    z = a_ref[0, :_N] + a_ref[1, :_N] + x_ref[...]
    o_ref[...] = jnp.dot(z, wl_ref[...], preferred_element_type=jnp.float32)


def _tc(body, out_shape, *args, interpret=False):
    return pl.pallas_call(body, out_shape=out_shape, interpret=interpret)(*args)


def _gin(x, edge_index, W0a, b0a, W0b, b0b, Wm1, bm1, Wm2, bm2, Wlast,
         interpret=False):
    src = edge_index[0]
    dst = edge_index[1]
    agg16 = _make_agg(16, interpret=interpret)
    agg32 = _make_agg(32, interpret=interpret)

    f32 = jnp.float32
    y = _tc(_proj0_body, jax.ShapeDtypeStruct((_N, 16), f32), x, W0a,
            interpret=interpret)
    a = agg16(y, src, dst)
    xc, y = _tc(
        _node0_body,
        (jax.ShapeDtypeStruct((_N, 32), f32), jax.ShapeDtypeStruct((_N, 16), f32)),
        a, y, b0a, W0b, b0b, Wm1[0],
        interpret=interpret,
    )
    for m in range(8):
        a = agg16(y, src, dst)
        if m < 7:
            xc, y = _tc(
                _node_mid_body,
                (jax.ShapeDtypeStruct((_N, 32), f32),
                 jax.ShapeDtypeStruct((_N, 16), f32)),
                xc, a, y, bm1[m], Wm2[m], bm2[m], Wm1[m + 1],
                interpret=interpret,
            )
        else:
            xc = _tc(
                _node_last_body,
                jax.ShapeDtypeStruct((_N, 32), f32),
                xc, a, y, bm1[m], Wm2[m], bm2[m],
                interpret=interpret,
            )
    a9 = agg32(xc, src, dst)
    out = _tc(_final_body, jax.ShapeDtypeStruct((_N, 128), f32), xc, a9, Wlast,
              interpret=interpret)
    return out


def kernel(x, edge_index, W0a, b0a, W0b, b0b, Wm1, bm1, Wm2, bm2, Wlast):
    return _gin(x, edge_index, W0a, b0a, W0b, b0b, Wm1, bm1, Wm2, bm2, Wlast)


# speedup vs baseline: 13.7831x; 1.8292x over previous
import functools

import jax
import jax.numpy as jnp
from jax import lax
from jax.experimental import pallas as pl
from jax.experimental.pallas import tpu as pltpu
from jax.experimental.pallas import tpu_sc as plsc

_N = 10000
_E = 320000
_NCORE = 2
_NSUB = 16
_NW = _NCORE * _NSUB
_CH = 128
_NCH = 80
_EPAD = _NW * _NCH * _CH
_GRP = 16
_NPAD = 10240
_RPT = _NPAD // _NSUB

_SELU_ALPHA = 1.6732632423543772
_SELU_SCALE = 1.0507009873554805


def _selu(v):
    return _SELU_SCALE * jnp.where(v > 0, v, _SELU_ALPHA * (jnp.exp(v) - 1.0))


def _make_agg(width, interpret=False):
    mesh = plsc.VectorSubcoreMesh(
        core_axis_name="c", subcore_axis_name="s",
        num_cores=_NCORE, num_subcores=_NSUB,
    )

    @functools.partial(
        pl.kernel,
        out_type=jax.ShapeDtypeStruct((_NCORE, _NPAD, width), jnp.float32),
        mesh=mesh,
        scratch_types=[
            pltpu.VMEM((_NCH, _CH), jnp.int32),
            pltpu.VMEM((_NCH, _CH), jnp.int32),
            pltpu.VMEM((_GRP, _CH, width), jnp.float32),
            pltpu.VMEM((_RPT, width), jnp.float32),
            pltpu.VMEM_SHARED((_NPAD, width), jnp.float32),
            pltpu.SemaphoreType.DMA,
            pltpu.SemaphoreType.DMA,
            pltpu.SemaphoreType.DMA,
        ],
        compiler_params=pltpu.CompilerParams(use_tc_tiling_on_sc=False),
        interpret=interpret,
    )
    def agg(y_hbm, src_hbm, dst_hbm, out_hbm, idx_s, idx_d, rows, zbuf, acc,
            isem, gsem, ssem):
        cid = lax.axis_index("c")
        sid = lax.axis_index("s")
        wid = sid * _NCORE + cid
        chunk0 = wid * _NCH

        pltpu.async_copy(src_hbm.at[pl.ds(chunk0, _NCH)], idx_s, isem)
        pltpu.async_copy(dst_hbm.at[pl.ds(chunk0, _NCH)], idx_d, isem)

        def zrow(j, carry):
            for w in range(width // 16):
                zbuf[j, pl.ds(w * 16, 16)] = jnp.zeros((16,), jnp.float32)
            return carry

        lax.fori_loop(0, _RPT, zrow, 0)
        row0 = sid * _RPT
        pltpu.sync_copy(zbuf, acc.at[pl.ds(row0, _RPT)])
        pltpu.make_async_copy(src_hbm.at[pl.ds(chunk0, _NCH)], idx_s, isem).wait()
        pltpu.make_async_copy(dst_hbm.at[pl.ds(chunk0, _NCH)], idx_d, isem).wait()
        plsc.subcore_barrier()

        def group(g, carry):
            j0 = g * _GRP
            for b in range(_GRP):
                pltpu.async_copy(
                    y_hbm.at[idx_s.at[j0 + b]], rows.at[b], gsem
                )
            for b in range(_GRP):
                pltpu.make_async_copy(
                    y_hbm.at[idx_s.at[j0 + b]], rows.at[b], gsem
                ).wait()
                pltpu.async_copy(
                    rows.at[b], acc.at[idx_d.at[j0 + b]], ssem, add=True
                )
            for b in range(_GRP):
                pltpu.make_async_copy(
                    rows.at[b], acc.at[idx_d.at[j0 + b]], ssem
                ).wait()
            return carry

        lax.fori_loop(0, _NCH // _GRP, group, 0)
        plsc.subcore_barrier()
        pltpu.sync_copy(
            acc.at[pl.ds(row0, _RPT)], out_hbm.at[cid, pl.ds(row0, _RPT)]
        )

    return agg


def _proj0_body(x_ref, w_ref, y_ref):
    y_ref[...] = jnp.dot(
        x_ref[...], w_ref[...], preferred_element_type=jnp.float32
    )


def _node0_body(a_ref, y_ref, b0a_ref, w0b_ref, b0b_ref, wm1_ref, x_ref, yn_ref):
    pre = a_ref[0, :_N] + a_ref[1, :_N] + y_ref[...] + b0a_ref[...]
    x1 = (
        jnp.dot(_selu(pre), w0b_ref[...], preferred_element_type=jnp.float32)
        + b0b_ref[...]
    )
    x_ref[...] = x1
    yn_ref[...] = jnp.dot(x1, wm1_ref[...], preferred_element_type=jnp.float32)


def _node_mid_body(
    x_ref, a_ref, y_ref, b1_ref, w2_ref, b2_ref, wn_ref, xo_ref, yn_ref
):
    pre = a_ref[0, :_N] + a_ref[1, :_N] + y_ref[...] + b1_ref[...]
    h = (
        jnp.dot(_selu(pre), w2_ref[...], preferred_element_type=jnp.float32)
        + b2_ref[...]
    )
    xn = x_ref[...] + h
    xo_ref[...] = xn
    yn_ref[...] = jnp.dot(xn, wn_ref[...], preferred_element_type=jnp.float32)


def _node_last_body(x_ref, a_ref, y_ref, b1_ref, w2_ref, b2_ref, xo_ref):
    pre = a_ref[0, :_N] + a_ref[1, :_N] + y_ref[...] + b1_ref[...]
    h = (
        jnp.dot(_selu(pre), w2_ref[...], preferred_element_type=jnp.float32)
        + b2_ref[...]
    )
    xo_ref[...] = x_ref[...] + h


def _final_body(x_ref, a_ref, wl_ref, o_ref):
    z = a_ref[0, :_N] + a_ref[1, :_N] + x_ref[...]
    o_ref[...] = jnp.dot(z, wl_ref[...], preferred_element_type=jnp.float32)


def _tc(body, out_shape, *args, interpret=False):
    return pl.pallas_call(body, out_shape=out_shape, interpret=interpret)(*args)


def _gin(x, edge_index, W0a, b0a, W0b, b0b, Wm1, bm1, Wm2, bm2, Wlast,
         interpret=False):
    pad = _EPAD - _E
    src = jnp.concatenate(
        [edge_index[0], jnp.zeros((pad,), jnp.int32)]
    ).reshape(_NW * _NCH, _CH)
    dst = jnp.concatenate(
        [edge_index[1], jnp.full((pad,), _N, jnp.int32)]
    ).reshape(_NW * _NCH, _CH)
    agg16 = _make_agg(16, interpret=interpret)
    agg32 = _make_agg(32, interpret=interpret)

    f32 = jnp.float32
    y = _tc(_proj0_body, jax.ShapeDtypeStruct((_N, 16), f32), x, W0a,
            interpret=interpret)
    a = agg16(y, src, dst)
    xc, y = _tc(
        _node0_body,
        (jax.ShapeDtypeStruct((_N, 32), f32), jax.ShapeDtypeStruct((_N, 16), f32)),
        a, y, b0a, W0b, b0b, Wm1[0],
        interpret=interpret,
    )
    for m in range(8):
        a = agg16(y, src, dst)
        if m < 7:
            xc, y = _tc(
                _node_mid_body,
                (jax.ShapeDtypeStruct((_N, 32), f32),
                 jax.ShapeDtypeStruct((_N, 16), f32)),
                xc, a, y, bm1[m], Wm2[m], bm2[m], Wm1[m + 1],
                interpret=interpret,
            )
        else:
            xc = _tc(
                _node_last_body,
                jax.ShapeDtypeStruct((_N, 32), f32),
                xc, a, y, bm1[m], Wm2[m], bm2[m],
                interpret=interpret,
            )
    a9 = agg32(xc, src, dst)
    out = _tc(_final_body, jax.ShapeDtypeStruct((_N, 128), f32), xc, a9, Wlast,
              interpret=interpret)
    return out


def kernel(x, edge_index, W0a, b0a, W0b, b0b, Wm1, bm1, Wm2, bm2, Wlast):
    return _gin(x, edge_index, W0a, b0a, W0b, b0b, Wm1, bm1, Wm2, bm2, Wlast)
